# Initial kernel scaffold; baseline (speedup 1.0000x reference)
import jax
import jax.numpy as jnp
from jax.experimental import pallas as pl


def kernel(emb, W1, b1, W2, b2, inputs):
    raise NotImplementedError("write your pallas kernel here")


# speedup vs baseline: 1.6619x; 1.6619x over previous
import functools

import jax
import jax.numpy as jnp
from jax import lax
from jax.experimental import pallas as pl
from jax.experimental.pallas import tpu as pltpu
from jax.experimental.pallas import tpu_sc as plsc

VOCAB = 100000
EMBEDDING_DIM = 128
MINIBATCH = 4096
NGRAM = 5
HDIM = 2048
TAGS = 1000

TOTAL_ROWS = MINIBATCH * NGRAM

try:
    _info = plsc.get_sparse_core_info()
    _NC, _NS = _info.num_cores, _info.num_subcores
except Exception:
    _NC, _NS = 2, 16
_NW = _NC * _NS
_ROWS_PER_W = TOTAL_ROWS // _NW
_IDX_CHUNK = 128
_N_CHUNKS = _ROWS_PER_W // _IDX_CHUNK


def _make_gather():
    mesh = plsc.VectorSubcoreMesh(core_axis_name="c", subcore_axis_name="s")

    @functools.partial(
        pl.kernel,
        mesh=mesh,
        out_type=jax.ShapeDtypeStruct((TOTAL_ROWS, EMBEDDING_DIM), jnp.float32),
        scratch_types=[
            pltpu.VMEM((_N_CHUNKS, _IDX_CHUNK), jnp.int32),
            pltpu.VMEM((_ROWS_PER_W, EMBEDDING_DIM), jnp.float32),
            pltpu.SemaphoreType.DMA,
        ],
    )
    def gather_k(idx_hbm, table_hbm, out_hbm, idx_v, rows_v, sem):
        wid = lax.axis_index("s") * _NC + lax.axis_index("c")
        base = wid * _ROWS_PER_W
        pltpu.sync_copy(idx_hbm.at[wid], idx_v)
        copies = []
        for j in range(_N_CHUNKS):
            copies.append(
                pltpu.async_copy(
                    table_hbm.at[idx_v.at[j]],
                    rows_v.at[pl.ds(j * _IDX_CHUNK, _IDX_CHUNK)],
                    sem,
                )
            )
        for c in copies:
            c.wait()
        pltpu.sync_copy(rows_v, out_hbm.at[pl.ds(base, _ROWS_PER_W)])

    return gather_k


_gather_cache = []


def _get_gather():
    if not _gather_cache:
        _gather_cache.append(_make_gather())
    return _gather_cache[0]


def _mlp_body(x_ref, w1_ref, b1_ref, w2_ref, b2_ref, o_ref):
    h = jnp.tanh(
        jnp.dot(x_ref[...], w1_ref[...], preferred_element_type=jnp.float32)
        + b1_ref[...]
    )
    o_ref[...] = (
        jnp.dot(h, w2_ref[...], preferred_element_type=jnp.float32) + b2_ref[...]
    )


_BM = 512


def _mlp(x, W1, b1, W2, b2):
    grid = (MINIBATCH // _BM,)
    return pl.pallas_call(
        _mlp_body,
        grid=grid,
        in_specs=[
            pl.BlockSpec((_BM, NGRAM * EMBEDDING_DIM), lambda i: (i, 0)),
            pl.BlockSpec((NGRAM * EMBEDDING_DIM, HDIM), lambda i: (0, 0)),
            pl.BlockSpec((1, HDIM), lambda i: (0, 0)),
            pl.BlockSpec((HDIM, TAGS), lambda i: (0, 0)),
            pl.BlockSpec((1, TAGS), lambda i: (0, 0)),
        ],
        out_specs=pl.BlockSpec((_BM, TAGS), lambda i: (i, 0)),
        out_shape=jax.ShapeDtypeStruct((MINIBATCH, TAGS), jnp.float32),
    )(x, W1, b1.reshape(1, HDIM), W2, b2.reshape(1, TAGS))


def kernel(emb, W1, b1, W2, b2, inputs):
    idx = inputs.astype(jnp.int32).reshape(_NW, _N_CHUNKS, _IDX_CHUNK)
    rows = _get_gather()(idx, emb)
    x = rows.reshape(MINIBATCH, NGRAM * EMBEDDING_DIM)
    return _mlp(x, W1, b1, W2, b2)
